# baseline (speedup 1.0000x reference)
import jax
import jax.numpy as jnp
from jax.experimental import pallas as pl
from jax.experimental.pallas import tpu as pltpu

N = 4096
D = 2048
H = 256
HEAD = 1024
CPAD = 128
BLK = 512
NI = N // BLK
NSIM = NI * (NI + 1) // 2
THRESH = 0.05

_BF = jnp.bfloat16


def _dot(a, b):
    return jax.lax.dot_general(
        a.astype(_BF), b.astype(_BF), (((1,), (0,)), ((), ())),
        preferred_element_type=jnp.float32)


def _gelu(v):
    return 0.5 * v * (1.0 + jax.lax.erf(v * (2.0 ** -0.5)))


def _layer_norm(v, g, b, eps=1e-5):
    mu = jnp.mean(v, axis=-1, keepdims=True)
    var = jnp.mean((v - mu) ** 2, axis=-1, keepdims=True)
    return (v - mu) * jax.lax.rsqrt(var + eps) * g + b


def _mega_kernel(x_ref, w0_ref, b0_ref, bh_ref, ln1g_ref, ln1b_ref,
                 ln2g_ref, ln2b_ref, wh_ref, bi_ref, bj_ref, h3_ref, cnt_ref,
                 fn_s, adj_s, p0_s, pb0_s, pb1_s, deg_s, dinv_s):
    t = pl.program_id(0)
    T_SIM = NI
    T_SCALE = NI + NSIM
    T_GCN = T_SCALE + 1

    @pl.when(t < NI)
    def _prep():
        x = x_ref[...]
        ninv = jax.lax.rsqrt(jnp.sum(x * x, axis=1, keepdims=True))
        fn_s[pl.ds(t * BLK, BLK), :] = (x * ninv).astype(_BF)
        p0_s[pl.ds(t * BLK, BLK), :] = _dot(x, w0_ref[...])
        deg_s[pl.ds(t * BLK, BLK), :] = jnp.zeros((BLK, 128), jnp.float32)

    @pl.when((t >= T_SIM) & (t < T_SCALE))
    def _sim():
        s = t - T_SIM
        bi = bi_ref[s]
        bj = bj_ref[s]
        fni = fn_s[pl.ds(bi * BLK, BLK), :]
        fnj = fn_s[pl.ds(bj * BLK, BLK), :]
        sim = jax.lax.dot_general(fni, fnj, (((1,), (1,)), ((), ())),
                                  preferred_element_type=jnp.float32)
        mask = (sim >= THRESH).astype(jnp.float32)
        adj_s[pl.ds(bi * BLK, BLK), pl.ds(bj * BLK, BLK)] = (
            mask.astype(jnp.int8))
        degi = jnp.sum(mask, axis=1, keepdims=True)
        deg_s[pl.ds(bi * BLK, BLK), :] = (
            deg_s[pl.ds(bi * BLK, BLK), :]
            + jnp.broadcast_to(degi, (BLK, 128)))

        @pl.when(bi != bj)
        def _mirror():
            ident = jnp.equal(
                jax.lax.broadcasted_iota(jnp.int32, (BLK, BLK), 0),
                jax.lax.broadcasted_iota(jnp.int32, (BLK, BLK), 1)
            ).astype(_BF)
            maskt = jax.lax.dot_general(
                mask.astype(_BF), ident, (((0,), (0,)), ((), ())),
                preferred_element_type=jnp.float32)
            adj_s[pl.ds(bj * BLK, BLK), pl.ds(bi * BLK, BLK)] = (
                maskt.astype(jnp.int8))
            degj = jnp.sum(maskt, axis=1, keepdims=True)
            deg_s[pl.ds(bj * BLK, BLK), :] = (
                deg_s[pl.ds(bj * BLK, BLK), :]
                + jnp.broadcast_to(degj, (BLK, 128)))

    @pl.when(t == T_SCALE)
    def _scale():
        deg = deg_s[...]
        cnt_ref[0, 0] = jnp.sum(deg[:, 0])
        dinv = jax.lax.rsqrt(deg)
        dinv_s[...] = dinv
        pb1_s[...] = (p0_s[...] * dinv[:, 0:1]).astype(_BF)

    def layer(i, pb_s, b_ref):
        dinv_i = dinv_s[pl.ds(i * BLK, BLK), 0:1]
        adji = adj_s[pl.ds(i * BLK, BLK), :]
        agg = jax.lax.dot_general(
            adji.astype(_BF), pb_s[...], (((1,), (0,)), ((), ())),
            preferred_element_type=jnp.float32)
        return agg * dinv_i + b_ref[...], dinv_i

    @pl.when((t >= T_GCN) & (t < T_GCN + NI))
    def _layer0():
        i = t - T_GCN
        a, dinv_i = layer(i, pb1_s, b0_ref)
        h = _gelu(_layer_norm(a, ln1g_ref[...], ln1b_ref[...]))
        pb0_s[pl.ds(i * BLK, BLK), :] = (_dot(h, wh_ref[...])
                                         * dinv_i).astype(_BF)

    @pl.when((t >= T_GCN + NI) & (t < T_GCN + 2 * NI))
    def _layer1():
        i = t - (T_GCN + NI)
        a, dinv_i = layer(i, pb0_s, bh_ref)
        h = _gelu(_layer_norm(a, ln2g_ref[...], ln2b_ref[...]))
        pb1_s[pl.ds(i * BLK, BLK), :] = (_dot(h, wh_ref[...])
                                         * dinv_i).astype(_BF)

    @pl.when(t >= T_GCN + 2 * NI)
    def _layer2():
        i = t - (T_GCN + 2 * NI)
        a, _ = layer(i, pb1_s, bh_ref)
        h3_ref[...] = a


def _graph_gcn(x, W0, b0, bh, ln1g, ln1b, ln2g, ln2b, Wh):
    pairs = [(i, j) for i in range(NI) for j in range(i, NI)]
    bi_arr = jnp.array([p[0] for p in pairs], jnp.int32)
    bj_arr = jnp.array([p[1] for p in pairs], jnp.int32)

    t_out = NI + NSIM + 1 + 2 * NI
    cvec = lambda: pl.BlockSpec((1, H), lambda t: (0, 0))
    h3, cnt = pl.pallas_call(
        _mega_kernel,
        grid=(NI + NSIM + 1 + 3 * NI,),
        in_specs=[
            pl.BlockSpec((BLK, D), lambda t: (jnp.minimum(t, NI - 1), 0)),
            pl.BlockSpec((D, H), lambda t: (0, 0)),
            cvec(), cvec(), cvec(), cvec(), cvec(), cvec(),
            pl.BlockSpec((H, H), lambda t: (0, 0)),
            pl.BlockSpec(memory_space=pltpu.SMEM),
            pl.BlockSpec(memory_space=pltpu.SMEM),
        ],
        out_specs=[
            pl.BlockSpec((BLK, H),
                         lambda t: (jnp.clip(t - t_out, 0, NI - 1), 0)),
            pl.BlockSpec(memory_space=pltpu.SMEM),
        ],
        out_shape=[
            jax.ShapeDtypeStruct((N, H), jnp.float32),
            jax.ShapeDtypeStruct((1, 1), jnp.float32),
        ],
        scratch_shapes=[
            pltpu.VMEM((N, D), _BF),
            pltpu.VMEM((N, N), jnp.int8),
            pltpu.VMEM((N, H), jnp.float32),
            pltpu.VMEM((N, H), _BF),
            pltpu.VMEM((N, H), _BF),
            pltpu.VMEM((N, 128), jnp.float32),
            pltpu.VMEM((N, 128), jnp.float32),
        ],
    )(x, W0.astype(_BF), b0, bh, ln1g, ln1b, ln2g, ln2b, Wh.astype(_BF),
      bi_arr, bj_arr)
    return h3, cnt


def _head_kernel(x_ref, h3_ref, hw1_ref, hb1_ref, g_ref, b_ref, hw2_ref,
                 hb2_ref, sw_ref, sb_ref, lm_ref, ls_ref):
    x = x_ref[...]
    z = (_dot(x, hw1_ref[0:D, :]) + _dot(h3_ref[...], hw1_ref[D:D + H, :])
         + hb1_ref[...])
    z = _gelu(z)
    z = _layer_norm(z, g_ref[...], b_ref[...])
    lm_ref[...] = _dot(z, hw2_ref[...]) + hb2_ref[...]
    ls_ref[...] = _dot(x, sw_ref[...]) + sb_ref[...]


def _heads(x, h3, hW1, hb1, g, b, hW2p, hb2p, sWp, sbp):
    return pl.pallas_call(
        _head_kernel,
        grid=(NI,),
        in_specs=[
            pl.BlockSpec((BLK, D), lambda i: (i, 0)),
            pl.BlockSpec((BLK, H), lambda i: (i, 0)),
            pl.BlockSpec((D + H, HEAD), lambda i: (0, 0)),
            pl.BlockSpec((1, HEAD), lambda i: (0, 0)),
            pl.BlockSpec((1, HEAD), lambda i: (0, 0)),
            pl.BlockSpec((1, HEAD), lambda i: (0, 0)),
            pl.BlockSpec((HEAD, CPAD), lambda i: (0, 0)),
            pl.BlockSpec((1, CPAD), lambda i: (0, 0)),
            pl.BlockSpec((D, CPAD), lambda i: (0, 0)),
            pl.BlockSpec((1, CPAD), lambda i: (0, 0)),
        ],
        out_specs=[pl.BlockSpec((BLK, CPAD), lambda i: (i, 0)),
                   pl.BlockSpec((BLK, CPAD), lambda i: (i, 0))],
        out_shape=[jax.ShapeDtypeStruct((N, CPAD), jnp.float32),
                   jax.ShapeDtypeStruct((N, CPAD), jnp.float32)],
    )(x.astype(_BF), h3, hW1.astype(_BF), hb1, g, b, hW2p.astype(_BF),
      hb2p, sWp.astype(_BF), sbp)


def kernel(x, W0, b0, Wh, bh, ln1_g, ln1_b, ln2_g, ln2_b, hW1, hb1,
           hln_g, hln_b, hW2, hb2, sW, sb):
    r = lambda v: v.reshape(1, -1)
    padc = lambda m: jnp.pad(m, ((0, 0), (0, CPAD - m.shape[1])))

    h3, cnt = _graph_gcn(x, W0, r(b0), r(bh), r(ln1_g), r(ln1_b),
                         r(ln2_g), r(ln2_b), Wh)
    lm, ls = _heads(x, h3, hW1, r(hb1), r(hln_g), r(hln_b), padc(hW2),
                    padc(r(hb2)), padc(sW), padc(r(sb)))

    logits_main = lm[:, :100]
    logits_second = ls[:, :100]
    density = (cnt[0, 0] * (1.0 / (N * N))).astype(jnp.float32)
    return (logits_main, logits_second, density)


# speedup vs baseline: 1.0392x; 1.0392x over previous
import jax
import jax.numpy as jnp
from jax.experimental import pallas as pl
from jax.experimental.pallas import tpu as pltpu

N = 4096
D = 2048
H = 256
HEAD = 1024
CPAD = 128
BLK = 512
NI = N // BLK
PBLK = 256
NP = N // PBLK
NSIM = NI * (NI + 1) // 2
THRESH = 0.05

_BF = jnp.bfloat16


def _dot(a, b):
    return jax.lax.dot_general(
        a.astype(_BF), b.astype(_BF), (((1,), (0,)), ((), ())),
        preferred_element_type=jnp.float32)


def _gelu(v):
    return 0.5 * v * (1.0 + jax.lax.erf(v * (2.0 ** -0.5)))


def _layer_norm(v, g, b, eps=1e-5):
    mu = jnp.mean(v, axis=-1, keepdims=True)
    var = jnp.mean((v - mu) ** 2, axis=-1, keepdims=True)
    return (v - mu) * jax.lax.rsqrt(var + eps) * g + b


def _mega_kernel(x_ref, w0_ref, b0_ref, bh_ref, ln1g_ref, ln1b_ref,
                 ln2g_ref, ln2b_ref, wh_ref, hw1_ref, hb1_ref, hg_ref,
                 hb_ref, hw2_ref, hb2_ref, sw_ref, sb_ref, bi_ref, bj_ref,
                 lm_ref, ls_ref, cnt_ref,
                 fn_s, adj_s, pb0_s, pb1_s, dd_s, nrm_s):
    t = pl.program_id(0)
    T_SIM = NP
    T_SCALE = T_SIM + NSIM
    T_GCN = T_SCALE + NI
    T_L2 = T_GCN + 2 * NI

    @pl.when(t < NP)
    def _prep():
        x = x_ref[...]
        nrm = jnp.sqrt(jnp.sum(x * x, axis=1, keepdims=True))
        nrm_s[pl.ds(t * PBLK, PBLK), :] = jnp.broadcast_to(nrm, (PBLK, 128))
        fn_s[pl.ds(t * PBLK, PBLK), :] = (x * (1.0 / nrm)).astype(_BF)
        dd_s[pl.ds(t * PBLK, PBLK), :] = jnp.zeros((PBLK, 128), jnp.float32)
        ls_ref[...] = _dot(x, sw_ref[...]) + sb_ref[...]

    @pl.when((t >= T_SIM) & (t < T_SCALE))
    def _sim():
        s = t - T_SIM
        bi = bi_ref[s]
        bj = bj_ref[s]
        fni = fn_s[pl.ds(bi * BLK, BLK), :]
        fnj = fn_s[pl.ds(bj * BLK, BLK), :]
        sim = jax.lax.dot_general(fni, fnj, (((1,), (1,)), ((), ())),
                                  preferred_element_type=jnp.float32)
        mask = (sim >= THRESH).astype(jnp.float32)
        adj_s[pl.ds(bi * BLK, BLK), pl.ds(bj * BLK, BLK)] = (
            mask.astype(jnp.int8))
        degi = jnp.sum(mask, axis=1, keepdims=True)
        dd_s[pl.ds(bi * BLK, BLK), :] = (
            dd_s[pl.ds(bi * BLK, BLK), :]
            + jnp.broadcast_to(degi, (BLK, 128)))

        @pl.when(bi != bj)
        def _mirror():
            ident = jnp.equal(
                jax.lax.broadcasted_iota(jnp.int32, (BLK, BLK), 0),
                jax.lax.broadcasted_iota(jnp.int32, (BLK, BLK), 1)
            ).astype(_BF)
            maskt = jax.lax.dot_general(
                mask.astype(_BF), ident, (((0,), (0,)), ((), ())),
                preferred_element_type=jnp.float32)
            adj_s[pl.ds(bj * BLK, BLK), pl.ds(bi * BLK, BLK)] = (
                maskt.astype(jnp.int8))
            degj = jnp.sum(maskt, axis=1, keepdims=True)
            dd_s[pl.ds(bj * BLK, BLK), :] = (
                dd_s[pl.ds(bj * BLK, BLK), :]
                + jnp.broadcast_to(degj, (BLK, 128)))

    @pl.when((t >= T_SCALE) & (t < T_GCN))
    def _scale():
        s = t - T_SCALE
        deg = dd_s[pl.ds(s * BLK, BLK), :]
        prev = jnp.where(s == 0, 0.0, cnt_ref[0, 0])
        cnt_ref[0, 0] = prev + jnp.sum(deg[:, 0])
        dinv = jax.lax.rsqrt(deg)
        dd_s[pl.ds(s * BLK, BLK), :] = dinv
        p0 = (_dot(fn_s[pl.ds(s * BLK, BLK), :], w0_ref[...])
              * nrm_s[pl.ds(s * BLK, BLK), 0:1])
        pb1_s[pl.ds(s * BLK, BLK), :] = (p0 * dinv[:, 0:1]).astype(_BF)

    def layer(i, pb_s, b_ref):
        dinv_i = dd_s[pl.ds(i * BLK, BLK), 0:1]
        adji = adj_s[pl.ds(i * BLK, BLK), :]
        agg = jax.lax.dot_general(
            adji.astype(_BF), pb_s[...], (((1,), (0,)), ((), ())),
            preferred_element_type=jnp.float32)
        return agg * dinv_i + b_ref[...], dinv_i

    @pl.when((t >= T_GCN) & (t < T_GCN + NI))
    def _layer0():
        i = t - T_GCN
        a, dinv_i = layer(i, pb1_s, b0_ref)
        h = _gelu(_layer_norm(a, ln1g_ref[...], ln1b_ref[...]))
        pb0_s[pl.ds(i * BLK, BLK), :] = (_dot(h, wh_ref[...])
                                         * dinv_i).astype(_BF)

    @pl.when((t >= T_GCN + NI) & (t < T_L2))
    def _layer1():
        i = t - (T_GCN + NI)
        a, dinv_i = layer(i, pb0_s, bh_ref)
        h = _gelu(_layer_norm(a, ln2g_ref[...], ln2b_ref[...]))
        pb1_s[pl.ds(i * BLK, BLK), :] = (_dot(h, wh_ref[...])
                                         * dinv_i).astype(_BF)

    @pl.when(t >= T_L2)
    def _layer2_head():
        i = t - T_L2
        h3, _ = layer(i, pb1_s, bh_ref)
        xb = (fn_s[pl.ds(i * BLK, BLK), :]
              * nrm_s[pl.ds(i * BLK, BLK), 0:1]).astype(_BF)
        z = (jax.lax.dot_general(xb, hw1_ref[0:D, :],
                                 (((1,), (0,)), ((), ())),
                                 preferred_element_type=jnp.float32)
             + _dot(h3, hw1_ref[D:D + H, :]) + hb1_ref[...])
        z = _gelu(z)
        z = _layer_norm(z, hg_ref[...], hb_ref[...])
        lm_ref[...] = _dot(z, hw2_ref[...]) + hb2_ref[...]


def _fused(x, W0, b0, bh, ln1g, ln1b, ln2g, ln2b, Wh, hW1, hb1, hg, hb,
           hW2p, hb2p, sWp, sbp):
    pairs = [(i, j) for i in range(NI) for j in range(i, NI)]
    bi_arr = jnp.array([p[0] for p in pairs], jnp.int32)
    bj_arr = jnp.array([p[1] for p in pairs], jnp.int32)

    t_l2 = NP + NSIM + NI + 2 * NI
    cvec = lambda n: pl.BlockSpec((1, n), lambda t: (0, 0))
    lm, ls, cnt = pl.pallas_call(
        _mega_kernel,
        grid=(t_l2 + NI,),
        in_specs=[
            pl.BlockSpec((PBLK, D), lambda t: (jnp.minimum(t, NP - 1), 0)),
            pl.BlockSpec((D, H), lambda t: (0, 0)),
            cvec(H), cvec(H), cvec(H), cvec(H), cvec(H), cvec(H),
            pl.BlockSpec((H, H), lambda t: (0, 0)),
            pl.BlockSpec((D + H, HEAD), lambda t: (0, 0)),
            cvec(HEAD), cvec(HEAD), cvec(HEAD),
            pl.BlockSpec((HEAD, CPAD), lambda t: (0, 0)),
            cvec(CPAD),
            pl.BlockSpec((D, CPAD), lambda t: (0, 0)),
            cvec(CPAD),
            pl.BlockSpec(memory_space=pltpu.SMEM),
            pl.BlockSpec(memory_space=pltpu.SMEM),
        ],
        out_specs=[
            pl.BlockSpec((BLK, CPAD),
                         lambda t: (jnp.clip(t - t_l2, 0, NI - 1), 0)),
            pl.BlockSpec((PBLK, CPAD),
                         lambda t: (jnp.minimum(t, NP - 1), 0)),
            pl.BlockSpec(memory_space=pltpu.SMEM),
        ],
        out_shape=[
            jax.ShapeDtypeStruct((N, CPAD), jnp.float32),
            jax.ShapeDtypeStruct((N, CPAD), jnp.float32),
            jax.ShapeDtypeStruct((1, 1), jnp.float32),
        ],
        scratch_shapes=[
            pltpu.VMEM((N, D), _BF),
            pltpu.VMEM((N, N), jnp.int8),
            pltpu.VMEM((N, H), _BF),
            pltpu.VMEM((N, H), _BF),
            pltpu.VMEM((N, 128), jnp.float32),
            pltpu.VMEM((N, 128), jnp.float32),
        ],
    )(x, W0.astype(_BF), b0, bh, ln1g, ln1b, ln2g, ln2b, Wh.astype(_BF),
      hW1.astype(_BF), hb1, hg, hb, hW2p.astype(_BF), hb2p,
      sWp.astype(_BF), sbp, bi_arr, bj_arr)
    return lm, ls, cnt


def kernel(x, W0, b0, Wh, bh, ln1_g, ln1_b, ln2_g, ln2_b, hW1, hb1,
           hln_g, hln_b, hW2, hb2, sW, sb):
    r = lambda v: v.reshape(1, -1)
    padc = lambda m: jnp.pad(m, ((0, 0), (0, CPAD - m.shape[1])))

    lm, ls, cnt = _fused(x, W0, r(b0), r(bh), r(ln1_g), r(ln1_b),
                         r(ln2_g), r(ln2_b), Wh, hW1, r(hb1), r(hln_g),
                         r(hln_b), padc(hW2), padc(r(hb2)), padc(sW),
                         padc(r(sb)))

    logits_main = lm[:, :100]
    logits_second = ls[:, :100]
    density = (cnt[0, 0] * (1.0 / (N * N))).astype(jnp.float32)
    return (logits_main, logits_second, density)
